# baseline (speedup 1.0000x reference)
import functools

import jax
import jax.numpy as jnp
from jax import lax
from jax.experimental import pallas as pl
from jax.experimental.pallas import tpu as pltpu
from jax.experimental.pallas import tpu_sc as plsc

_R = 16
_U = 8


def _sc_kernel(B, S, D):
    info = plsc.get_sparse_core_info()
    NW = info.num_cores * info.num_subcores
    L = info.num_lanes
    s_per_w = S // NW
    nblk = s_per_w // _R
    ncol = D // L
    mesh = plsc.VectorSubcoreMesh(core_axis_name="c", subcore_axis_name="s")

    @functools.partial(
        pl.kernel,
        mesh=mesh,
        out_type=jax.ShapeDtypeStruct((B, S, D), jnp.float32),
        scratch_types=[
            pltpu.VMEM((4, _R, D), jnp.float32),
            pltpu.VMEM((2, _R, D), jnp.float32),
            pltpu.VMEM((_R, D), jnp.float32),
            pltpu.SemaphoreType.DMA,
            pltpu.SemaphoreType.DMA,
            pltpu.SemaphoreType.DMA,
            pltpu.SemaphoreType.DMA,
            pltpu.SemaphoreType.DMA,
            pltpu.SemaphoreType.DMA,
        ],
    )
    def k(x_hbm, pe_hbm, out_hbm, pe_v, x_v, o_v, sp0, sp1, sx0, sx1, so0, so1):
        wid = lax.axis_index("s") * info.num_cores + lax.axis_index("c")
        s_base = wid * s_per_w
        spe = (sp0, sp1)
        sx = (sx0, sx1)
        sout = (so0, so1)

        def fire_pe(sblk, q, par):
            s0 = s_base + sblk * _R
            pltpu.async_copy(pe_hbm.at[pl.ds(s0, _R)], pe_v.at[q], spe[par])

        def drain_pe(q, sem):
            pltpu.make_async_copy(
                pe_hbm.at[pl.ds(s_base, _R)], pe_v.at[q], sem
            ).wait()

        def fire_x(sblk, b, xb):
            s0 = s_base + sblk * _R
            pltpu.async_copy(x_hbm.at[b, pl.ds(s0, _R)], x_v.at[xb], sx[xb])

        def drain_x(xb):
            pltpu.make_async_copy(
                x_hbm.at[0, pl.ds(s_base, _R)], x_v.at[xb], sx[xb]
            ).wait()

        def drain_out(q, sem):
            for b in range(B):
                pltpu.make_async_copy(
                    pe_v.at[q], out_hbm.at[b, pl.ds(s_base, _R)], sem
                ).wait()

        def scan_unit(xb):
            def col(cu, acc):
                for u in range(_U):
                    k_ = cu * _U + u
                    r, c = k_ // ncol, k_ % ncol
                    xv = x_v[xb, r, pl.ds(c * L, L)]
                    xi = lax.bitcast_convert_type(xv, jnp.int32)
                    acc = jnp.minimum(acc, xi & jnp.int32(0x7FFFFFFF))
                return acc

            acc = lax.fori_loop(
                0, (_R * ncol) // _U, col, jnp.full((L,), 1, jnp.int32)
            )
            zero = acc[0] == 0
            for l in range(1, L):
                zero = jnp.logical_or(zero, acc[l] == 0)
            return zero

        def rewrite_unit(b, q, xb, s0):
            def row(r, carry):
                def col(c, carry2):
                    xv = x_v[xb, r, pl.ds(c * L, L)]
                    pv = pe_v[q, r, pl.ds(c * L, L)]
                    o_v[r, pl.ds(c * L, L)] = jnp.where(xv == 0.0, 0.0, pv)
                    return carry2

                return lax.fori_loop(0, ncol, col, carry)

            lax.fori_loop(0, _R, row, None)
            pltpu.sync_copy(o_v, out_hbm.at[b, pl.ds(s0, _R)])

        def body(sblk, q, prev_fast):
            par = q % 2
            s0 = s_base + sblk * _R
            drain_pe(q, spe[par])

            @pl.when(prev_fast)
            def _():
                drain_out((q + 2) % 4, sout[par])

            for b in range(B):
                pltpu.async_copy(pe_v.at[q], out_hbm.at[b, pl.ds(s0, _R)], sout[par])

            @pl.when(sblk + 2 < nblk)
            def _():
                fire_pe(sblk + 2, (q + 2) % 4, par)

            drained = jnp.bool_(False)
            for b in range(B):
                xb = b % 2
                if b < B - 1:
                    fire_x(sblk, b + 1, (b + 1) % 2)
                else:

                    @pl.when(sblk + 1 < nblk)
                    def _():
                        fire_x(sblk + 1, 0, (b + 1) % 2)

                drain_x(xb)
                zb = scan_unit(xb)

                @pl.when(zb)
                def _(b=b, xb=xb, drained=drained):
                    @pl.when(jnp.logical_not(drained))
                    def _():
                        drain_out(q, sout[par])

                    rewrite_unit(b, q, xb, s0)

                drained = jnp.logical_or(drained, zb)

            return jnp.logical_not(drained)

        def step(j, carry):
            fA, fB = carry
            flags = [fA, fB]
            for q_ in range(4):
                sblk = 4 * j + q_
                fast = body(sblk, q_, flags[q_])
                flags.append(fast)
            return flags[4], flags[5]

        fire_pe(0, 0, 0)
        fire_pe(1, 1, 1)
        fire_x(0, 0, 0)
        f = jnp.bool_(False)
        fA, fB = lax.fori_loop(0, nblk // 4, step, (f, f))

        @pl.when(fA)
        def _():
            drain_out((nblk - 2) % 4, sout[0])

        @pl.when(fB)
        def _():
            drain_out((nblk - 1) % 4, sout[1])

    return k


def kernel(x, position_enc):
    B, S, D = x.shape
    pe = position_enc[:S]
    return _sc_kernel(B, S, D)(x, pe)


# speedup vs baseline: 1.0736x; 1.0736x over previous
import functools

import jax
import jax.numpy as jnp
from jax import lax
from jax.experimental import pallas as pl
from jax.experimental.pallas import tpu as pltpu
from jax.experimental.pallas import tpu_sc as plsc

_R = 8
_U = 8


def _sc_kernel(B, S, D):
    info = plsc.get_sparse_core_info()
    NW = info.num_cores * info.num_subcores
    L = info.num_lanes
    s_per_w = S // NW
    nblk = s_per_w // _R
    ncol = D // L
    mesh = plsc.VectorSubcoreMesh(core_axis_name="c", subcore_axis_name="s")

    @functools.partial(
        pl.kernel,
        mesh=mesh,
        out_type=jax.ShapeDtypeStruct((B, S, D), jnp.float32),
        scratch_types=[
            pltpu.VMEM((4, _R, D), jnp.float32),
            pltpu.VMEM((2, B, _R, D), jnp.float32),
            pltpu.VMEM((_R, D), jnp.float32),
            pltpu.SemaphoreType.DMA,
            pltpu.SemaphoreType.DMA,
            pltpu.SemaphoreType.DMA,
            pltpu.SemaphoreType.DMA,
            pltpu.SemaphoreType.DMA,
            pltpu.SemaphoreType.DMA,
        ],
    )
    def k(x_hbm, pe_hbm, out_hbm, pe_v, x_v, o_v, sp0, sp1, sx0, sx1, so0, so1):
        wid = lax.axis_index("s") * info.num_cores + lax.axis_index("c")
        s_base = wid * s_per_w
        spe = (sp0, sp1)
        sx = (sx0, sx1)
        sout = (so0, so1)

        def fire_in(blk_idx, p, q):
            s0 = s_base + blk_idx * _R
            pltpu.async_copy(pe_hbm.at[pl.ds(s0, _R)], pe_v.at[q], spe[p])
            for b in range(B):
                pltpu.async_copy(x_hbm.at[b, pl.ds(s0, _R)], x_v.at[p, b], sx[p])

        def drain_pe(p, q):
            pltpu.make_async_copy(
                pe_hbm.at[pl.ds(s_base, _R)], pe_v.at[q], spe[p]
            ).wait()

        def drain_x(p):
            for b in range(B):
                pltpu.make_async_copy(
                    x_hbm.at[b, pl.ds(s_base, _R)], x_v.at[p, b], sx[p]
                ).wait()

        def drain_out(q, sem):
            for b in range(B):
                pltpu.make_async_copy(
                    pe_v.at[q], out_hbm.at[b, pl.ds(s_base, _R)], sem
                ).wait()

        def body(i, p, q, prev_fast):
            s0 = s_base + i * _R
            drain_pe(p, q)

            @pl.when(prev_fast)
            def _():
                drain_out((q + 2) % 4, sout[p])

            for b in range(B):
                pltpu.async_copy(pe_v.at[q], out_hbm.at[b, pl.ds(s0, _R)], sout[p])

            drain_x(p)

            def scan_batch(b, acc0):
                def col(cu, acc):
                    for u in range(_U):
                        k_ = cu * _U + u
                        r, c = k_ // ncol, k_ % ncol
                        xv = x_v[p, b, r, pl.ds(c * L, L)]
                        xi = lax.bitcast_convert_type(xv, jnp.int32)
                        acc = jnp.minimum(acc, xi & jnp.int32(0x7FFFFFFF))
                    return acc

                return lax.fori_loop(0, (_R * ncol) // _U, col, acc0)

            acc = jnp.full((L,), 1, jnp.int32)
            for b in range(B):
                acc = scan_batch(b, acc)
            block_zero = acc[0] == 0
            for l in range(1, L):
                block_zero = jnp.logical_or(block_zero, acc[l] == 0)
            fast = jnp.logical_not(block_zero)

            @pl.when(block_zero)
            def _():
                drain_out(q, sout[p])
                for b in range(B):
                    def row(r, carry):
                        def col(c, carry2):
                            xv = x_v[p, b, r, pl.ds(c * L, L)]
                            pv = pe_v[q, r, pl.ds(c * L, L)]
                            o_v[r, pl.ds(c * L, L)] = jnp.where(xv == 0.0, 0.0, pv)
                            return carry2

                        return lax.fori_loop(0, ncol, col, carry)

                    lax.fori_loop(0, _R, row, None)
                    pltpu.sync_copy(o_v, out_hbm.at[b, pl.ds(s0, _R)])

            return fast

        def step(j, carry):
            fA, fB = carry
            flags = [fA, fB]
            for p_ in range(4):
                i = 4 * j + p_
                p = p_ % 2
                fast = body(i, p, p_, flags[p_])
                flags.append(fast)
                if p_ < 2:
                    fire_in(i + 2, p, (p_ + 2) % 4)
                else:

                    @pl.when(j < nblk // 4 - 1)
                    def _():
                        fire_in(i + 2, p, (p_ + 2) % 4)

            return flags[4], flags[5]

        fire_in(0, 0, 0)
        fire_in(1, 1, 1)
        f = jnp.bool_(False)
        fA, fB = lax.fori_loop(0, nblk // 4, step, (f, f))

        @pl.when(fA)
        def _():
            drain_out((nblk - 2) % 4, sout[0])

        @pl.when(fB)
        def _():
            drain_out((nblk - 1) % 4, sout[1])

    return k


def kernel(x, position_enc):
    B, S, D = x.shape
    pe = position_enc[:S]
    return _sc_kernel(B, S, D)(x, pe)
